# baseline (speedup 1.0000x reference)
import functools

import jax
import jax.numpy as jnp
from jax import lax
from jax.experimental import pallas as pl
from jax.experimental.pallas import tpu as pltpu
from jax.experimental.pallas import tpu_sc as plsc

DIM = 64
LANES = 16
CHUNK = 128

_GDN = lax.GatherDimensionNumbers(
    offset_dims=(), collapsed_slice_dims=(0,), start_index_map=(0,))


def _permute(v, idx):
    return lax.gather(v, idx[:, None], _GDN, (1,),
                      mode=lax.GatherScatterMode.PROMISE_IN_BOUNDS)


@functools.lru_cache(maxsize=None)
def _build(B, n_entities, n_relations, nc, ns):
    nw = nc * ns
    b_per_w = B // nw
    n_chunks = b_per_w // CHUNK
    mesh = plsc.VectorSubcoreMesh(core_axis_name="c", subcore_axis_name="s")

    @functools.partial(
        pl.kernel,
        mesh=mesh,
        compiler_params=pltpu.CompilerParams(use_tc_tiling_on_sc=False),
        out_type=jax.ShapeDtypeStruct((B,), jnp.float32),
        scratch_types=[
            pltpu.VMEM((n_chunks, CHUNK), jnp.int32),
            pltpu.VMEM((n_chunks, CHUNK), jnp.int32),
            pltpu.VMEM((n_chunks, CHUNK), jnp.int32),
            pltpu.VMEM((b_per_w, DIM), jnp.float32),
            pltpu.VMEM((b_per_w, DIM), jnp.float32),
            pltpu.VMEM((b_per_w, DIM), jnp.float32),
            pltpu.VMEM((b_per_w,), jnp.float32),
            pltpu.SemaphoreType.DMA,
        ],
    )
    def dist_mult(e_hbm, r_hbm, hi_hbm, ri_hbm, ti_hbm, out_hbm,
                  idx_h, idx_r, idx_t, h_rows, r_rows, t_rows, scores, sem):
        wid = lax.axis_index("s") * nc + lax.axis_index("c")

        pltpu.sync_copy(hi_hbm.at[pl.ds(wid * n_chunks, n_chunks)], idx_h)
        pltpu.sync_copy(ri_hbm.at[pl.ds(wid * n_chunks, n_chunks)], idx_r)
        pltpu.sync_copy(ti_hbm.at[pl.ds(wid * n_chunks, n_chunks)], idx_t)

        copies = []
        for j in range(n_chunks):
            dst = pl.ds(j * CHUNK, CHUNK)
            copies.append(pltpu.async_copy(e_hbm.at[idx_h.at[j]], h_rows.at[dst], sem))
            copies.append(pltpu.async_copy(r_hbm.at[idx_r.at[j]], r_rows.at[dst], sem))
            copies.append(pltpu.async_copy(e_hbm.at[idx_t.at[j]], t_rows.at[dst], sem))
        for c in copies:
            c.wait()

        lane = lax.broadcasted_iota(jnp.int32, (LANES,), 0)
        perms = [lane ^ k for k in (8, 4, 2, 1)]

        def group(g, carry):
            vec = jnp.zeros((LANES,), jnp.float32)
            for rloc in range(LANES):
                i = g * LANES + rloc
                acc = jnp.zeros((LANES,), jnp.float32)
                for cb in range(DIM // LANES):
                    cs = pl.ds(cb * LANES, LANES)
                    acc = acc + h_rows[i, cs] * r_rows[i, cs] * t_rows[i, cs]
                for p in perms:
                    acc = acc + _permute(acc, p)
                vec = jnp.where(lane == rloc, acc, vec)
            scores[pl.ds(g * LANES, LANES)] = vec
            return carry

        lax.fori_loop(0, b_per_w // LANES, group, 0)

        pltpu.sync_copy(scores, out_hbm.at[pl.ds(wid * b_per_w, b_per_w)])

    return dist_mult


def kernel(h_idx, r_idx, t_idx, E, R):
    B = h_idx.shape[0]
    info = plsc.get_sparse_core_info()
    nc, ns = info.num_cores, info.num_subcores
    nw = nc * ns
    n_chunks = (B // nw) // CHUNK
    f = _build(B, E.shape[0], R.shape[0], nc, ns)
    h2 = h_idx.astype(jnp.int32).reshape(nw * n_chunks, CHUNK)
    r2 = r_idx.astype(jnp.int32).reshape(nw * n_chunks, CHUNK)
    t2 = t_idx.astype(jnp.int32).reshape(nw * n_chunks, CHUNK)
    return f(E, R, h2, r2, t2)


# speedup vs baseline: 1.6451x; 1.6451x over previous
import functools

import jax
import jax.numpy as jnp
from jax import lax
from jax.experimental import pallas as pl
from jax.experimental.pallas import tpu as pltpu
from jax.experimental.pallas import tpu_sc as plsc

DIM = 64
LANES = 16
CH = 32

_GDN = lax.GatherDimensionNumbers(
    offset_dims=(), collapsed_slice_dims=(0,), start_index_map=(0,))


def _permute(v, idx):
    return lax.gather(v, idx[:, None], _GDN, (1,),
                      mode=lax.GatherScatterMode.PROMISE_IN_BOUNDS)


@functools.lru_cache(maxsize=None)
def _build(B, n_entities, n_relations, nc, ns):
    nw = nc * ns
    b_per_w = B // nw
    n_chunks = b_per_w // CH
    mesh = plsc.VectorSubcoreMesh(core_axis_name="c", subcore_axis_name="s")

    @functools.partial(
        pl.kernel,
        mesh=mesh,
        out_type=jax.ShapeDtypeStruct((B,), jnp.float32),
        scratch_types=[
            pltpu.VMEM((b_per_w,), jnp.int32),
            pltpu.VMEM((b_per_w,), jnp.int32),
            pltpu.VMEM((b_per_w,), jnp.int32),
            pltpu.VMEM((2, CH, DIM), jnp.float32),
            pltpu.VMEM((2, CH, DIM), jnp.float32),
            pltpu.VMEM((2, CH, DIM), jnp.float32),
            pltpu.VMEM((b_per_w,), jnp.float32),
            pltpu.SemaphoreType.DMA,
            pltpu.SemaphoreType.DMA,
        ],
    )
    def dist_mult(e_hbm, r_hbm, hi_hbm, ri_hbm, ti_hbm, out_hbm,
                  idx_h, idx_r, idx_t, h_rows, r_rows, t_rows, scores,
                  sem0, sem1):
        wid = lax.axis_index("s") * nc + lax.axis_index("c")
        base = wid * b_per_w

        pltpu.sync_copy(hi_hbm.at[pl.ds(base, b_per_w)], idx_h)
        pltpu.sync_copy(ri_hbm.at[pl.ds(base, b_per_w)], idx_r)
        pltpu.sync_copy(ti_hbm.at[pl.ds(base, b_per_w)], idx_t)

        sems = [sem0, sem1]

        def fire(c, buf):
            sem = sems[buf]
            cps = []
            for g in range(CH // LANES):
                hv = idx_h[pl.ds(c * CH + g * LANES, LANES)]
                rv = idx_r[pl.ds(c * CH + g * LANES, LANES)]
                tv = idx_t[pl.ds(c * CH + g * LANES, LANES)]
                for k in range(LANES):
                    i = g * LANES + k
                    cps.append(pltpu.async_copy(e_hbm.at[pl.ds(hv[k], 1)],
                                                h_rows.at[buf, pl.ds(i, 1)], sem))
                    cps.append(pltpu.async_copy(r_hbm.at[pl.ds(rv[k], 1)],
                                                r_rows.at[buf, pl.ds(i, 1)], sem))
                    cps.append(pltpu.async_copy(e_hbm.at[pl.ds(tv[k], 1)],
                                                t_rows.at[buf, pl.ds(i, 1)], sem))
            return cps

        def drain(cps):
            for c in cps:
                c.wait()

        lane = lax.broadcasted_iota(jnp.int32, (LANES,), 0)
        perms = [lane ^ k for k in (8, 4, 2, 1)]

        def compute(c, buf):
            for g in range(CH // LANES):
                vec = jnp.zeros((LANES,), jnp.float32)
                for rloc in range(LANES):
                    i = g * LANES + rloc
                    acc = jnp.zeros((LANES,), jnp.float32)
                    for cb in range(DIM // LANES):
                        cs = pl.ds(cb * LANES, LANES)
                        acc = (acc + h_rows[buf, i, cs] * r_rows[buf, i, cs]
                               * t_rows[buf, i, cs])
                    for p in perms:
                        acc = acc + _permute(acc, p)
                    vec = jnp.where(lane == rloc, acc, vec)
                scores[pl.ds(c * CH + g * LANES, LANES)] = vec

        def step(c, carry):
            cps = fire(c, 0)
            drain(cps)
            compute(c, 0)
            return carry

        lax.fori_loop(0, n_chunks, step, 0)

        pltpu.sync_copy(scores, out_hbm.at[pl.ds(base, b_per_w)])

    return dist_mult


def kernel(h_idx, r_idx, t_idx, E, R):
    B = h_idx.shape[0]
    info = plsc.get_sparse_core_info()
    f = _build(B, E.shape[0], R.shape[0], info.num_cores, info.num_subcores)
    return f(E, R, h_idx.astype(jnp.int32), r_idx.astype(jnp.int32),
             t_idx.astype(jnp.int32))


# speedup vs baseline: 1.6467x; 1.0009x over previous
import functools

import jax
import jax.numpy as jnp
from jax import lax
from jax.experimental import pallas as pl
from jax.experimental.pallas import tpu as pltpu
from jax.experimental.pallas import tpu_sc as plsc

DIM = 64
LANES = 16
CH = 32

_GDN = lax.GatherDimensionNumbers(
    offset_dims=(), collapsed_slice_dims=(0,), start_index_map=(0,))


def _permute(v, idx):
    return lax.gather(v, idx[:, None], _GDN, (1,),
                      mode=lax.GatherScatterMode.PROMISE_IN_BOUNDS)


@functools.lru_cache(maxsize=None)
def _build(B, n_entities, n_relations, nc, ns):
    nw = nc * ns
    b_per_w = B // nw
    n_chunks = b_per_w // CH
    mesh = plsc.VectorSubcoreMesh(core_axis_name="c", subcore_axis_name="s")

    @functools.partial(
        pl.kernel,
        mesh=mesh,
        compiler_params=pltpu.CompilerParams(use_tc_tiling_on_sc=True),
        out_type=jax.ShapeDtypeStruct((B,), jnp.float32),
        scratch_types=[
            pltpu.VMEM((b_per_w,), jnp.int32),
            pltpu.VMEM((b_per_w,), jnp.int32),
            pltpu.VMEM((b_per_w,), jnp.int32),
            pltpu.VMEM((2, CH, DIM), jnp.float32),
            pltpu.VMEM((2, CH, DIM), jnp.float32),
            pltpu.VMEM((2, CH, DIM), jnp.float32),
            pltpu.VMEM((b_per_w,), jnp.float32),
            pltpu.SemaphoreType.DMA,
            pltpu.SemaphoreType.DMA,
        ],
    )
    def dist_mult(e_hbm, r_hbm, hi_hbm, ri_hbm, ti_hbm, out_hbm,
                  idx_h, idx_r, idx_t, h_rows, r_rows, t_rows, scores,
                  sem0, sem1):
        wid = lax.axis_index("s") * nc + lax.axis_index("c")
        base = wid * b_per_w

        pltpu.sync_copy(hi_hbm.at[pl.ds(base, b_per_w)], idx_h)
        pltpu.sync_copy(ri_hbm.at[pl.ds(base, b_per_w)], idx_r)
        pltpu.sync_copy(ti_hbm.at[pl.ds(base, b_per_w)], idx_t)

        sems = [sem0, sem1]

        def fire(c, buf):
            sem = sems[buf]
            cps = []
            for g in range(CH // LANES):
                hv = idx_h[pl.ds(c * CH + g * LANES, LANES)]
                rv = idx_r[pl.ds(c * CH + g * LANES, LANES)]
                tv = idx_t[pl.ds(c * CH + g * LANES, LANES)]
                for k in range(LANES):
                    i = g * LANES + k
                    cps.append(pltpu.async_copy(e_hbm.at[pl.ds(hv[k], 1)],
                                                h_rows.at[buf, pl.ds(i, 1)], sem))
                    cps.append(pltpu.async_copy(r_hbm.at[pl.ds(rv[k], 1)],
                                                r_rows.at[buf, pl.ds(i, 1)], sem))
                    cps.append(pltpu.async_copy(e_hbm.at[pl.ds(tv[k], 1)],
                                                t_rows.at[buf, pl.ds(i, 1)], sem))
            return cps

        def drain(cps):
            for c in cps:
                c.wait()

        lane = lax.broadcasted_iota(jnp.int32, (LANES,), 0)
        perms = [lane ^ k for k in (8, 4, 2, 1)]

        def compute(c, buf):
            for g in range(CH // LANES):
                vec = jnp.zeros((LANES,), jnp.float32)
                for rloc in range(LANES):
                    i = g * LANES + rloc
                    acc = jnp.zeros((LANES,), jnp.float32)
                    for cb in range(DIM // LANES):
                        cs = pl.ds(cb * LANES, LANES)
                        acc = (acc + h_rows[buf, i, cs] * r_rows[buf, i, cs]
                               * t_rows[buf, i, cs])
                    for p in perms:
                        acc = acc + _permute(acc, p)
                    vec = jnp.where(lane == rloc, acc, vec)
                scores[pl.ds(c * CH + g * LANES, LANES)] = vec

        def step(c, carry):
            cps = fire(c, 0)
            drain(cps)
            compute(c, 0)
            return carry

        lax.fori_loop(0, n_chunks, step, 0)

        pltpu.sync_copy(scores, out_hbm.at[pl.ds(base, b_per_w)])

    return dist_mult


def kernel(h_idx, r_idx, t_idx, E, R):
    B = h_idx.shape[0]
    info = plsc.get_sparse_core_info()
    f = _build(B, E.shape[0], R.shape[0], info.num_cores, info.num_subcores)
    return f(E, R, h_idx.astype(jnp.int32), r_idx.astype(jnp.int32),
             t_idx.astype(jnp.int32))
